# Initial kernel scaffold; baseline (speedup 1.0000x reference)
import jax
import jax.numpy as jnp
from jax.experimental import pallas as pl


def kernel(user_emb, item_emb, edge_play_ori, edge_play_csr, W1, b1, W2, b2):
    raise NotImplementedError("write your pallas kernel here")


# speedup vs baseline: 2.1812x; 2.1812x over previous
import dataclasses
import functools

import jax
import jax.numpy as jnp
from jax import lax
from jax.experimental import pallas as pl
from jax.experimental.pallas import tpu as pltpu
from jax.experimental.pallas import tpu_sc as plsc

N = 10000
D = 256
DH = 128
NE = 160000
NP = 10240
ACC_ROWS = 10112
ROWS_PT = 632
HB2 = 80
NB = 2
CHUNK = 128
CPT = 80
EPAD = 163840
TRASH = 10016
BR = 256
BR2 = 1264

def _mesh():
    return plsc.VectorSubcoreMesh(core_axis_name="c", subcore_axis_name="s")


def _pad_edges(e, fill):
    pad = jnp.full((EPAD - NE,), fill, jnp.int32)
    return jnp.concatenate([e.astype(jnp.int32), pad]).reshape(
        EPAD // CHUNK, CHUNK)


def _hist_body(edges, zeros_hbm, deg_out, idxv, ridx, sb, hist):
    c = lax.axis_index("c")
    s = lax.axis_index("s")
    iota = lax.iota(jnp.int32, 16)
    ones_v = jnp.full((16,), 1.0, jnp.float32)
    zeros_v = jnp.zeros((16,), jnp.float32)
    pltpu.sync_copy(zeros_hbm, sb)
    for side in range(2):
        @pl.when(s < 10)
        def _():
            pltpu.sync_copy(zeros_hbm.at[pl.ds(0, 8)],
                            hist.at[pl.ds(s * 8, 8)])

        pltpu.sync_copy(edges.at[c, side, pl.ds(s * CPT, CPT)], idxv)

        @pl.loop(0, CPT)
        def _(j):
            for u in range(8):
                b = idxv[j, pl.ds(u * 16, 16)]
                ridx[j, pl.ds(u * 16, 16)] = lax.shift_right_logical(b, 7)

        plsc.subcore_barrier()

        @pl.loop(0, CPT)
        def _(j):
            for u in range(8):
                b = idxv[j, pl.ds(u * 16, 16)]
                rows = iota + (u * 16)
                lanes = lax.bitwise_and(b, 127)
                plsc.store_scatter(sb, [rows, lanes], ones_v)
            pltpu.sync_copy(sb, hist.at[ridx.at[j]], add=True)
            for u in range(8):
                b = idxv[j, pl.ds(u * 16, 16)]
                rows = iota + (u * 16)
                lanes = lax.bitwise_and(b, 127)
                plsc.store_scatter(sb, [rows, lanes], zeros_v)

        plsc.subcore_barrier()

        @pl.when(s < 10)
        def _():
            pltpu.sync_copy(hist.at[pl.ds(s * 8, 8)],
                            deg_out.at[c, side, pl.ds(s * 8, 8)])

        if side == 0:
            plsc.subcore_barrier()


def _no_layout_passes():
    cp = pltpu.CompilerParams()
    if "needs_layout_passes" in pltpu.CompilerParams.__dataclass_fields__:
        cp = dataclasses.replace(cp, needs_layout_passes=False)
    return cp


def _sc_hist(edges):
    zerosh = jnp.zeros((CHUNK, CHUNK), jnp.float32)
    fn = pl.kernel(
        _hist_body,
        mesh=_mesh(),
        compiler_params=_no_layout_passes(),
        out_type=jax.ShapeDtypeStruct((2, 2, HB2, CHUNK), jnp.float32),
        scratch_types=[
            pltpu.VMEM((CPT, CHUNK), jnp.int32),
            pltpu.VMEM((CPT, CHUNK), jnp.int32),
            pltpu.VMEM((CHUNK, CHUNK), jnp.float32),
            pltpu.VMEM_SHARED((HB2, CHUNK), jnp.float32),
        ],
    )
    return fn(edges, zerosh)


def _scale_body(x_ref, dsrc_ref, xs_ref):
    r = lax.rsqrt(jnp.maximum(dsrc_ref[0, 0, :, 0:1], 1.0))
    y = x_ref[0] * r
    xs_ref[0, 0] = y[:, :DH]
    xs_ref[0, 1] = y[:, DH:]


def _tc_scale(xstk, deg):
    return pl.pallas_call(
        _scale_body,
        grid=(4, NP // BR),
        in_specs=[
            pl.BlockSpec((1, BR, D), lambda k, i: (k % 2, i, 0)),
            pl.BlockSpec((1, 1, BR, 16), lambda k, i: (k // 2, k % 2, i, 0)),
        ],
        out_specs=pl.BlockSpec((1, 2, BR, DH), lambda k, i: (k, 0, i, 0)),
        out_shape=jax.ShapeDtypeStruct((4, 2, NP, DH), jnp.float32),
    )(xstk, deg)


EPC = EPAD // CHUNK


def _agg_body(xs, goff, sidx, zeros_hbm, agg_out, gv, sv, rb, acc):
    c = lax.axis_index("c")
    s = lax.axis_index("s")
    r0 = s * ROWS_PT

    @pl.loop(0, 4)
    def _(k):
        pltpu.sync_copy(zeros_hbm, acc.at[pl.ds(r0, ROWS_PT)])
        pltpu.sync_copy(goff.at[pl.ds((k * 2 + c) * EPC + s * CPT, CPT)], gv)
        pltpu.sync_copy(sidx.at[pl.ds(k * EPC + s * CPT, CPT)], sv)
        plsc.subcore_barrier()

        @pl.loop(0, CPT)
        def _(j):
            pltpu.sync_copy(xs.at[gv.at[j]], rb)
            pltpu.sync_copy(rb, acc.at[sv.at[j]], add=True)

        plsc.subcore_barrier()
        pltpu.sync_copy(
            acc.at[pl.ds(r0, ROWS_PT)],
            agg_out.at[pl.ds((k * 2 + c) * ACC_ROWS + r0, ROWS_PT)])


def _sc_agg(xs_flat, goff, sidx):
    zerosh = jnp.zeros((ROWS_PT, DH), jnp.float32)
    fn = pl.kernel(
        _agg_body,
        mesh=_mesh(),
        out_type=jax.ShapeDtypeStruct((4 * 2 * ACC_ROWS, DH), jnp.float32),
        scratch_types=[
            pltpu.VMEM((CPT, CHUNK), jnp.int32),
            pltpu.VMEM((CPT, CHUNK), jnp.int32),
            pltpu.VMEM((CHUNK, DH), jnp.float32),
            pltpu.VMEM_SHARED((ACC_ROWS, DH), jnp.float32),
        ],
    )
    return fn(xs_flat, goff, sidx, zerosh)


def _mm_body(agg_ref, w_ref, b_ref, dd_ref, o_ref):
    a = jnp.concatenate([agg_ref[0, 0], agg_ref[0, 1]], axis=1)
    y = jnp.dot(a, w_ref[0], preferred_element_type=jnp.float32)
    r = lax.rsqrt(jnp.maximum(dd_ref[0, 0, :, 0:1], 1.0))
    o_ref[0] = y * r + b_ref[0]


def _tc_matmul(agg, wstk, bstk, deg):
    return pl.pallas_call(
        _mm_body,
        grid=(4, ACC_ROWS // BR2),
        in_specs=[
            pl.BlockSpec((1, 2, BR2, DH), lambda k, i: (k, 0, i, 0)),
            pl.BlockSpec((1, D, D), lambda k, i: (k, 0, 0)),
            pl.BlockSpec((1, 1, D), lambda k, i: (k, 0, 0)),
            pl.BlockSpec((1, 1, BR2, 16),
                         lambda k, i: (k // 2, 1 - k % 2, i, 0)),
        ],
        out_specs=pl.BlockSpec((1, BR2, D), lambda k, i: (k, i, 0)),
        out_shape=jax.ShapeDtypeStruct((4, ACC_ROWS, D), jnp.float32),
    )(agg, wstk, bstk, deg)


def kernel(user_emb, item_emb, edge_play_ori, edge_play_csr, W1, b1, W2, b2):
    so = _pad_edges(edge_play_ori[0], TRASH)
    do = _pad_edges(edge_play_ori[1], TRASH)
    sc = _pad_edges(edge_play_csr[0], TRASH)
    dc = _pad_edges(edge_play_csr[1], TRASH)
    so0 = _pad_edges(edge_play_ori[0], 0)
    do0 = _pad_edges(edge_play_ori[1], 0)
    sc0 = _pad_edges(edge_play_csr[0], 0)
    dc0 = _pad_edges(edge_play_csr[1], 0)

    edges_hist = jnp.stack(
        [jnp.stack([so, do]), jnp.stack([sc, dc])])
    deg2 = _sc_hist(edges_hist)
    deg = jnp.broadcast_to(deg2.reshape(2, 2, NP, 1),
                           (2, 2, NP, 16))

    xstk = jnp.stack([
        jnp.pad(user_emb, ((0, NP - N), (0, 0))),
        jnp.pad(item_emb, ((0, NP - N), (0, 0))),
    ])
    xs4 = _tc_scale(xstk, deg)

    gidx = jnp.stack([so, do, sc, dc])
    off = (2 * jnp.arange(4, dtype=jnp.int32)[:, None]
           + jnp.arange(2, dtype=jnp.int32)[None, :]) * NP
    goff = gidx[:, None] + off[:, :, None, None]
    sidx = jnp.stack([do0, so0, dc0, sc0])

    agg = _sc_agg(xs4.reshape(4 * 2 * NP, DH),
                  goff.reshape(4 * 2 * EPC, CHUNK),
                  sidx.reshape(4 * EPC, CHUNK))
    agg = agg.reshape(4, 2, ACC_ROWS, DH)

    wstk = jnp.stack([W1, W1, W2, W2])
    bstk = jnp.stack([b1, b1, b2, b2]).reshape(4, 1, D)
    out = _tc_matmul(agg, wstk, bstk, deg[:, :, :ACC_ROWS])

    return (out[0, :N], out[1, :N], out[2, :N], out[3, :N])
